# baseline (speedup 1.0000x reference)
import jax
import jax.numpy as jnp
from jax import lax
from jax.experimental import pallas as pl
from jax.experimental.pallas import tpu as pltpu
from jax.experimental.pallas import tpu_sc as plsc

N_DRUG, N_CELL, N_GENE = 1000, 500, 3000
NN = N_DRUG + N_CELL + N_GENE
EE = NN * 32
EFULL = EE + NN
HEADS, H2, H3 = 8, 128, 128
DD = HEADS * H2

WIN = 592
NWIN = 16
SHARD = WIN * NWIN
PAD_E = SHARD * 16
PAD_DST = 1 << 30

NRANGE = 4
RNG = 1152
SROWS = RNG + 16
CHROWS = 16
SHARD32 = PAD_E // 32
NW32 = SHARD32 // WIN
N0PAD = 4608
SENT0 = 4600

_MESH = plsc.VectorSubcoreMesh(core_axis_name="c", subcore_axis_name="s")


def _bcast(vec16, i):
    return vec16.at[jnp.full((16,), i, jnp.int32)].get(
        mode='promise_in_bounds')
_SC_PARAMS = pltpu.CompilerParams(needs_layout_passes=False)
_TC_PARAMS = pltpu.CompilerParams(vmem_limit_bytes=110 * 2**20)


def _matmul_bias_body(x_ref, w_ref, b_ref, o_ref):
    o_ref[...] = jnp.dot(x_ref[...], w_ref[...],
                         preferred_element_type=jnp.float32) + b_ref[...]


def _matmul_bias(x, w, b):
    m = x.shape[0]
    n = w.shape[1]
    return pl.pallas_call(
        _matmul_bias_body,
        out_shape=jax.ShapeDtypeStruct((m, n), jnp.float32),
    )(x, w, b[None, :])


def _matmul_body_nobias(x_ref, w_ref, o_ref):
    o_ref[...] = jnp.dot(x_ref[...], w_ref[...],
                         preferred_element_type=jnp.float32)


def _t2_body(x_ref, w_ref, asf_ref, adf_ref, wef_ref, aef_ref, ea_ref,
             xp_ref, tab_ref, wc_ref):
    x = x_ref[...]
    W = w_ref[...]
    xp_ref[...] = jnp.dot(x, W, preferred_element_type=jnp.float32)
    ki = lax.broadcasted_iota(jnp.int32, (DD, HEADS), 0)
    hi = lax.broadcasted_iota(jnp.int32, (DD, HEADS), 1)
    M = (ki // H2 == hi).astype(jnp.float32)
    Was = jnp.dot(W * asf_ref[...], M, precision=lax.Precision.HIGHEST,
                  preferred_element_type=jnp.float32)
    Wad = jnp.dot(W * adf_ref[...], M, precision=lax.Precision.HIGHEST,
                  preferred_element_type=jnp.float32)
    a_src = jnp.dot(x, Was, precision=lax.Precision.HIGHEST,
                    preferred_element_type=jnp.float32)
    a_dst = jnp.dot(x, Wad, precision=lax.Precision.HIGHEST,
                    preferred_element_type=jnp.float32)
    tab_ref[...] = jnp.concatenate([a_src, a_dst], axis=1)
    w8 = jnp.dot(wef_ref[...] * aef_ref[...], M,
                 precision=lax.Precision.HIGHEST,
                 preferred_element_type=jnp.float32)
    ea = ea_ref[...]
    ea_max = jnp.maximum(jnp.max(ea), 0.0)
    ea_min = jnp.minimum(jnp.min(ea), 0.0)
    ub = (jnp.max(a_src, axis=0, keepdims=True)
          + jnp.max(a_dst, axis=0, keepdims=True)
          + jnp.maximum(w8 * ea_max, w8 * ea_min))
    C = jnp.where(ub > 0, ub, 0.2 * ub)
    row = jnp.concatenate([w8, C], axis=1)
    wc_ref[...] = jnp.broadcast_to(row, (8, 16))


def _t2(x, W, asf, adf, wef, aef, ea2d):
    n = x.shape[0]
    return pl.pallas_call(
        _t2_body,
        compiler_params=_TC_PARAMS,
        out_shape=(
            jax.ShapeDtypeStruct((n, DD), jnp.float32),
            jax.ShapeDtypeStruct((n, 16), jnp.float32),
            jax.ShapeDtypeStruct((8, 16), jnp.float32),
        ),
    )(x, W, asf[None, :], adf[None, :], wef[None, :], aef[None, :], ea2d)


def _t3a_body(asum_ref, scale_ref, asc_ref):
    a = asum_ref[...]
    ji = lax.broadcasted_iota(jnp.int32, (HEADS, 32 * HEADS), 1)
    hi = lax.broadcasted_iota(jnp.int32, (HEADS, 32 * HEADS), 0)
    S = (ji % HEADS == hi).astype(jnp.float32)
    tot = jnp.dot(S, a, precision=lax.Precision.HIGHEST,
                  preferred_element_type=jnp.float32)
    ssum = jnp.sum(tot / (tot + 1e-16))
    inv = 1.0 / (tot + 1e-16)
    hi2 = lax.broadcasted_iota(jnp.int32, (HEADS, DD), 0)
    ci2 = lax.broadcasted_iota(jnp.int32, (HEADS, DD), 1)
    M2 = (ci2 // H2 == hi2).astype(jnp.float32)
    scale_ref[...] = lax.dot_general(
        inv, M2, (((0,), (0,)), ((), ())),
        precision=lax.Precision.HIGHEST,
        preferred_element_type=jnp.float32)
    asc_ref[...] = jnp.broadcast_to(ssum.reshape(1, 1), (8, 16))


def _t3b_body(outp_ref, scale_ref, b_ref, nw_ref, nb_ref, nms_ref, x_ref):
    y = outp_ref[...][:NN] * scale_ref[...][:NN] + b_ref[...]
    mean = jnp.mean(y, axis=0, keepdims=True)
    out = y - nms_ref[...] * mean
    var = jnp.mean(out * out, axis=0, keepdims=True)
    x_ref[...] = jnp.maximum(
        out / jnp.sqrt(var + 1e-5) * nw_ref[...] + nb_ref[...], 0.0)


_CBLK = 256


def _t3(outp, asum2d, b, nw, nb, nms):
    scale, asc = pl.pallas_call(
        _t3a_body,
        out_shape=(
            jax.ShapeDtypeStruct((N0PAD, DD), jnp.float32),
            jax.ShapeDtypeStruct((8, 16), jnp.float32),
        ),
    )(asum2d)
    x = pl.pallas_call(
        _t3b_body,
        grid=(DD // _CBLK,),
        in_specs=[
            pl.BlockSpec((N0PAD, _CBLK), lambda i: (0, i)),
            pl.BlockSpec((N0PAD, _CBLK), lambda i: (0, i)),
            pl.BlockSpec((1, _CBLK), lambda i: (0, i)),
            pl.BlockSpec((1, _CBLK), lambda i: (0, i)),
            pl.BlockSpec((1, _CBLK), lambda i: (0, i)),
            pl.BlockSpec((1, _CBLK), lambda i: (0, i)),
        ],
        out_specs=pl.BlockSpec((NN, _CBLK), lambda i: (0, i)),
        out_shape=jax.ShapeDtypeStruct((NN, DD), jnp.float32),
    )(outp, scale, b[None, :], nw[None, :], nb[None, :], nms[None, :])
    return x, asc


def _tla_body(s_ref, c_ref, o_ref):
    ssum = jnp.sum(s_ref[...], axis=0, keepdims=True)
    csum = jnp.sum(c_ref[...], axis=0, keepdims=True)
    o_ref[...] = ssum / jnp.maximum(csum, 1.0)


def _tla(sums, cnts):
    return pl.pallas_call(
        _tla_body,
        out_shape=jax.ShapeDtypeStruct((1, N0PAD), jnp.float32),
    )(sums, cnts)


def _s0_body(dst_hbm, ea_hbm, parts_hbm, sums_p, cnts_p, meta_d, meta_e,
             sem):
    cid = lax.axis_index("c")
    sid = lax.axis_index("s")

    @pl.when(cid == 0)
    def _():
        def zi(i, _):
            sums_p[pl.ds(i * 16, 16)] = jnp.zeros((16,), jnp.float32)
            cnts_p[pl.ds(i * 16, 16)] = jnp.zeros((16,), jnp.float32)
            return 0
        lax.fori_loop(0, N0PAD // 16, zi, 0)

        def win_loop(w, _):
            wbase = sid * SHARD + w * WIN
            pltpu.sync_copy(dst_hbm.at[pl.ds(wbase, WIN)], meta_d)
            pltpu.sync_copy(ea_hbm.at[pl.ds(wbase, WIN)], meta_e)

            def chunk(c, _):
                epos = wbase + c * 16 + lax.iota(jnp.int32, 16)
                d16 = meta_d[pl.ds(c * 16, 16)]
                e16 = meta_e[pl.ds(c * 16, 16)]
                ok = epos < EE
                idx = jnp.where(ok, jnp.minimum(d16, SENT0), SENT0)
                plsc.addupdate_scatter(sums_p, [idx],
                                       jnp.where(ok, e16, 0.0))
                plsc.addupdate_scatter(cnts_p, [idx],
                                       jnp.where(ok, 1.0, 0.0))
                return 0
            lax.fori_loop(0, WIN // 16, chunk, 0)
            return 0
        lax.fori_loop(0, NWIN, win_loop, 0)

        pltpu.sync_copy(sums_p,
                        parts_hbm.at[pl.ds(sid * 2 * N0PAD, N0PAD)])
        pltpu.sync_copy(cnts_p,
                        parts_hbm.at[pl.ds(sid * 2 * N0PAD + N0PAD, N0PAD)])


def _s0(dstp, eap0):
    return pl.kernel(
        _s0_body,
        out_type=jax.ShapeDtypeStruct((16 * 2 * N0PAD,), jnp.float32),
        mesh=_MESH,
        compiler_params=_SC_PARAMS,
        scratch_types=[
            pltpu.VMEM((N0PAD,), jnp.float32),
            pltpu.VMEM((N0PAD,), jnp.float32),
            pltpu.VMEM((WIN,), jnp.int32),
            pltpu.VMEM((WIN,), jnp.float32),
            pltpu.SemaphoreType.DMA,
        ],
    )(dstp, eap0)


def _s1a_body(src_hbm, dst_hbm, ea_hbm, tab_hbm, wc_hbm,
              p_hbm, asum_hbm,
              tab, wc_v, meta_s, meta_d, meta_e, pwin, asum_p, sem):
    cid = lax.axis_index("c")
    sid = lax.axis_index("s")
    wid = sid * 2 + cid
    pltpu.sync_copy(tab_hbm, tab)
    pltpu.sync_copy(wc_hbm, wc_v)
    wrow = wc_v[0]

    def za(i, _):
        asum_p[pl.ds(i * 16, 16)] = jnp.zeros((16,), jnp.float32)
        return 0
    lax.fori_loop(0, (8 * N0PAD) // 16, za, 0)

    def win_loop(w, _):
        wbase = wid * SHARD32 + w * WIN
        pltpu.sync_copy(src_hbm.at[pl.ds(wbase, WIN)], meta_s)
        pltpu.sync_copy(dst_hbm.at[pl.ds(wbase, WIN)], meta_d)
        pltpu.sync_copy(ea_hbm.at[pl.ds(wbase, WIN)], meta_e)

        def chunk(c, _):
            epos = wbase + c * 16 + lax.iota(jnp.int32, 16)
            ok = epos < EFULL
            s16 = jnp.clip(meta_s[pl.ds(c * 16, 16)], 0, NN - 1)
            d16r = meta_d[pl.ds(c * 16, 16)]
            d16 = jnp.clip(d16r, 0, NN - 1)
            e16 = meta_e[pl.ds(c * 16, 16)]
            dsum = jnp.minimum(d16r, SENT0)
            for h in range(HEADS):
                as_h = plsc.load_gather(tab, [s16 * 16 + h])
                ad_h = plsc.load_gather(tab, [d16 * 16 + 8 + h])
                z = as_h + ad_h + e16 * _bcast(wrow, h)
                z = jnp.where(z > 0, z, 0.2 * z)
                p = jnp.exp(z - _bcast(wrow, 8 + h))
                p = jnp.where(ok, p, 0.0)
                plsc.addupdate_scatter(asum_p, [h * N0PAD + dsum], p)
                pwin[pl.ds(h * WIN + c * 16, 16)] = p
            return 0
        lax.fori_loop(0, WIN // 16, chunk, 0)
        for h in range(HEADS):
            pltpu.sync_copy(pwin.at[pl.ds(h * WIN, WIN)],
                            p_hbm.at[pl.ds(h * PAD_E + wbase, WIN)])
        return 0
    lax.fori_loop(0, NW32, win_loop, 0)
    pltpu.sync_copy(asum_p,
                    asum_hbm.at[pl.ds(wid * 8 * N0PAD, 8 * N0PAD)])


def _s1a(srcp, dstp, eap, tabflat, wc):
    return pl.kernel(
        _s1a_body,
        out_type=(
            jax.ShapeDtypeStruct((PAD_E * 8,), jnp.float32),
            jax.ShapeDtypeStruct((32 * 8 * N0PAD,), jnp.float32),
        ),
        mesh=_MESH,
        compiler_params=_SC_PARAMS,
        scratch_types=[
            pltpu.VMEM((NN * 16,), jnp.float32),
            pltpu.VMEM((8, 16), jnp.float32),
            pltpu.VMEM((WIN,), jnp.int32),
            pltpu.VMEM((WIN,), jnp.int32),
            pltpu.VMEM((WIN,), jnp.float32),
            pltpu.VMEM((WIN * 8,), jnp.float32),
            pltpu.VMEM((8 * N0PAD,), jnp.float32),
            pltpu.SemaphoreType.DMA,
        ],
    )(srcp, dstp, eap, tabflat, wc)


SEG = 2368


def _s1b_body(src_hbm, xp_hbm, rows_hbm, meta_s, gsrc, rowbuf, sem):
    cid = lax.axis_index("c")
    sid = lax.axis_index("s")
    wid = sid * 2 + cid

    for seg in range(SHARD32 // SEG):
        segbase = wid * SHARD32 + seg * SEG
        pltpu.sync_copy(src_hbm.at[pl.ds(segbase, SEG)], meta_s)

        def chunk(c, _):
            for q in range(4):
                j0 = c * 64 + q * 16
                s16 = jnp.clip(meta_s[pl.ds(j0, 16)], 0, NN - 1)
                gsrc[pl.ds(q * 16, 16)] = s16
            pltpu.async_copy(xp_hbm.at[gsrc], rowbuf, sem).wait()
            pltpu.sync_copy(rowbuf,
                            rows_hbm.at[pl.ds(segbase + c * 64, 64)])
            return 0
        lax.fori_loop(0, SEG // 64, chunk, 0)


def _s1b(srcp, xp):
    return pl.kernel(
        _s1b_body,
        out_type=jax.ShapeDtypeStruct((PAD_E, DD), jnp.float32),
        mesh=_MESH,
        compiler_params=_SC_PARAMS,
        scratch_types=[
            pltpu.VMEM((SEG,), jnp.int32),
            pltpu.VMEM((64,), jnp.int32),
            pltpu.VMEM((64, DD), jnp.float32),
            pltpu.SemaphoreType.DMA,
        ],
    )(srcp, xp)


EBLK = 256
NEB = PAD_E // EBLK


def _t4agg_body(dst_ref, p_ref, rows_ref, out_ref):
    i = pl.program_id(0)

    @pl.when(i == 0)
    def _():
        out_ref[...] = jnp.zeros(out_ref.shape, jnp.float32)

    dv = dst_ref[0, 0]
    ni = lax.broadcasted_iota(jnp.int32, (EBLK, N0PAD), 1)
    oneh = (dv[:, None] == ni).astype(jnp.float32)
    hi2 = lax.broadcasted_iota(jnp.int32, (HEADS, DD), 0)
    ci2 = lax.broadcasted_iota(jnp.int32, (HEADS, DD), 1)
    M2 = (ci2 // H2 == hi2).astype(jnp.float32)
    smat = lax.dot_general(p_ref[...], M2, (((0,), (0,)), ((), ())),
                           precision=lax.Precision.HIGHEST,
                           preferred_element_type=jnp.float32)
    rb = rows_ref[...] * smat
    out_ref[...] += lax.dot_general(
        oneh, rb, (((0,), (0,)), ((), ())),
        precision=lax.Precision.HIGHEST,
        preferred_element_type=jnp.float32)


def _t4agg(dst3, p8, rows):
    return pl.pallas_call(
        _t4agg_body,
        compiler_params=_TC_PARAMS,
        grid=(NEB,),
        in_specs=[
            pl.BlockSpec((1, 1, EBLK), lambda i: (i, 0, 0)),
            pl.BlockSpec((HEADS, EBLK), lambda i: (0, i)),
            pl.BlockSpec((EBLK, DD), lambda i: (i, 0)),
        ],
        out_specs=pl.BlockSpec((N0PAD, DD), lambda i: (0, 0)),
        out_shape=jax.ShapeDtypeStruct((N0PAD, DD), jnp.float32),
    )(dst3, p8, rows)


def _att_body(src_hbm, dst_hbm, mrow_hbm, out_hbm,
              zbuf, meta_s, meta_d, idxbuf, constbuf, mrow_v, sem):
    cid = lax.axis_index("c")
    sid = lax.axis_index("s")
    half = NN // 2
    row_lo = cid * half

    flat_base = row_lo * NN
    flat_len = half * NN
    base_len = (flat_len // 16) & ~7
    extra = flat_len - base_len * 16

    ZCH = zbuf.shape[0]

    def zinit(i, _):
        zbuf[pl.ds(i * 16, 16)] = jnp.zeros((16,), jnp.float32)
        return 0
    lax.fori_loop(0, ZCH // 16, zinit, 0)

    tstart = flat_base + sid * base_len
    nfull = base_len // ZCH
    rem = base_len - nfull * ZCH

    def zloop(i, _):
        pltpu.sync_copy(zbuf, out_hbm.at[pl.ds(tstart + i * ZCH, ZCH)])
        return 0
    lax.fori_loop(0, nfull, zloop, 0)
    if rem:
        pltpu.sync_copy(zbuf.at[pl.ds(0, rem)],
                        out_hbm.at[pl.ds(tstart + nfull * ZCH, rem)])
    if extra:
        @pl.when(sid == 0)
        def _():
            pltpu.sync_copy(zbuf.at[pl.ds(0, extra)],
                            out_hbm.at[pl.ds(flat_base + 16 * base_len, extra)])

    plsc.subcore_barrier()

    pltpu.sync_copy(mrow_hbm, mrow_v)
    mvec = mrow_v[0]
    dummy = (row_lo * NN) + row_lo
    shard_base = sid * SHARD

    def win_loop(w, _):
        wbase = shard_base + w * WIN
        pltpu.sync_copy(src_hbm.at[pl.ds(wbase, WIN)], meta_s)
        pltpu.sync_copy(dst_hbm.at[pl.ds(wbase, WIN)], meta_d)

        def chunk(c, _):
            s16 = meta_s[pl.ds(c * 16, 16)]
            d16 = meta_d[pl.ds(c * 16, 16)]
            ok = jnp.logical_and(
                d16 < NN,
                jnp.logical_and(s16 >= row_lo, s16 < row_lo + half))
            flat = jnp.where(ok, s16 * NN + d16, dummy)
            idxbuf[pl.ds(c * 16, 16)] = flat
            constbuf[pl.ds(c * 16, 16)] = mvec
            return 0
        lax.fori_loop(0, WIN // 16, chunk, 0)
        pltpu.sync_copy(constbuf, out_hbm.at[idxbuf])
        return 0
    lax.fori_loop(0, NWIN, win_loop, 0)


def _att_scatter(srcp, dstp, mrow):
    return pl.kernel(
        _att_body,
        out_type=jax.ShapeDtypeStruct((NN * NN,), jnp.float32),
        mesh=_MESH,
        compiler_params=_SC_PARAMS,
        scratch_types=[
            pltpu.VMEM((16384,), jnp.float32),
            pltpu.VMEM((WIN,), jnp.int32),
            pltpu.VMEM((WIN,), jnp.int32),
            pltpu.VMEM((WIN,), jnp.int32),
            pltpu.VMEM((WIN,), jnp.float32),
            pltpu.VMEM((8, 16), jnp.float32),
            pltpu.SemaphoreType.DMA,
        ],
    )(srcp, dstp, mrow)


B = 1024
_BPW = B // 32


def _fin_body(y_hbm, idxd_hbm, idxc_hbm, brow_hbm, out_hbm,
              ytab, idxd_v, idxc_v, out_v, brow_v, sem):
    cid = lax.axis_index("c")
    sid = lax.axis_index("s")
    wid = sid * 2 + cid
    base = wid * _BPW
    pltpu.sync_copy(brow_hbm, brow_v)
    bias = brow_v[0]
    pltpu.sync_copy(y_hbm, ytab)
    pltpu.sync_copy(idxd_hbm.at[pl.ds(base, _BPW)], idxd_v)
    pltpu.sync_copy(idxc_hbm.at[pl.ds(base, _BPW)], idxc_v)

    def chunk(c, _):
        d16 = idxd_v[pl.ds(c * 16, 16)]
        c16 = idxc_v[pl.ds(c * 16, 16)]
        yd = plsc.load_gather(ytab, [d16 * 16])
        yc = plsc.load_gather(ytab, [c16 * 16 + 1])
        out_v[pl.ds(c * 16, 16)] = yd + yc + bias
        return 0
    lax.fori_loop(0, _BPW // 16, chunk, 0)
    pltpu.sync_copy(out_v, out_hbm.at[pl.ds(base, _BPW)])


def _final_gather(y_flat, idx_drug, idx_cell, brow):
    return pl.kernel(
        _fin_body,
        out_type=jax.ShapeDtypeStruct((B,), jnp.float32),
        mesh=_MESH,
        compiler_params=_SC_PARAMS,
        scratch_types=[
            pltpu.VMEM((NN * 16,), jnp.float32),
            pltpu.VMEM((_BPW,), jnp.int32),
            pltpu.VMEM((_BPW,), jnp.int32),
            pltpu.VMEM((_BPW,), jnp.float32),
            pltpu.VMEM((8, 16), jnp.float32),
            pltpu.SemaphoreType.DMA,
        ],
    )(y_flat, idx_drug, idx_cell, brow)


def kernel(drug, cell, gene, edge_attr, edge_index, idx_drug, idx_cell,
           params):
    p = params
    x0 = jnp.concatenate([
        _matmul_bias(drug, p['Wd'], p['bd']),
        _matmul_bias(cell, p['Wc'], p['bc']),
        _matmul_bias(gene, p['Wg'], p['bg']),
    ], axis=0)
    ea0 = edge_attr.astype(jnp.float32)
    loop = jnp.arange(NN, dtype=jnp.int32)
    src = jnp.concatenate([edge_index[0].astype(jnp.int32), loop])
    dst = jnp.concatenate([edge_index[1].astype(jnp.int32), loop])
    npad = PAD_E - EFULL
    srcp = jnp.concatenate([src, jnp.zeros((npad,), jnp.int32)])
    dstp = jnp.concatenate([dst, jnp.full((npad,), PAD_DST, jnp.int32)])
    eap0 = jnp.concatenate([ea0, jnp.zeros((PAD_E - EE,), jnp.float32)])

    parts = _s0(dstp, eap0).reshape(16, 2, N0PAD)
    la = _tla(parts[:, 0], parts[:, 1])[0, :NN]
    dst3 = dstp.reshape(PAD_E // EBLK, 1, EBLK)
    eap = jnp.concatenate([ea0, la, jnp.zeros((npad,), jnp.float32)])
    ea2d = ea0.reshape(1125, 128)

    def layer(x, pre):
        asf = p[pre + '_as'].reshape(DD)
        adf = p[pre + '_ad'].reshape(DD)
        wef = p[pre + '_We'].reshape(DD)
        aef = p[pre + '_ae'].reshape(DD)
        xp, tab, wc = _t2(x, p[pre + '_W'], asf, adf, wef, aef, ea2d)
        pfull, asumf = _s1a(srcp, dstp, eap, tab.reshape(NN * 16), wc)
        rows = _s1b(srcp, xp)
        outp = _t4agg(dst3, pfull.reshape(8, PAD_E), rows)
        return outp, asumf.reshape(32 * 8, N0PAD)

    outp1, asum1 = layer(x0, 'g1')
    x1, ss1 = _t3(outp1, asum1, p['g1_b'], p['n1_w'], p['n1_b'], p['n1_ms'])
    outp2, asum2 = layer(x1, 'g2')
    x2, ss2 = _t3(outp2, asum2, p['g2_b'], p['n2_w'], p['n2_b'], p['n2_ms'])

    m = (ss1[0, 0] + ss2[0, 0]) / (EFULL * HEADS)
    att = _att_scatter(srcp, dstp, jnp.full((8, 16), m, jnp.float32))
    att = att.reshape(NN, NN)

    HH = HEADS * H3
    L2 = jnp.zeros((HH, 16), jnp.float32)
    L2 = L2.at[:, 0].set(p['l1_W'][:HH, 0])
    L2 = L2.at[:, 1].set(p['l1_W'][HH:, 0])
    y = pl.pallas_call(
        _matmul_body_nobias,
        out_shape=jax.ShapeDtypeStruct((NN, 16), jnp.float32),
    )(x2, L2)
    brow = jnp.full((8, 16), p['l1_b'][0], jnp.float32)
    out = _final_gather(y.reshape(NN * 16), idx_drug.astype(jnp.int32),
                        idx_cell.astype(jnp.int32), brow)
    return out[:, None], att


# speedup vs baseline: 1.0879x; 1.0879x over previous
import jax
import jax.numpy as jnp
from jax import lax
from jax.experimental import pallas as pl
from jax.experimental.pallas import tpu as pltpu
from jax.experimental.pallas import tpu_sc as plsc

N_DRUG, N_CELL, N_GENE = 1000, 500, 3000
NN = N_DRUG + N_CELL + N_GENE
EE = NN * 32
EFULL = EE + NN
HEADS, H2, H3 = 8, 128, 128
DD = HEADS * H2

WIN = 592
NWIN = 16
SHARD = WIN * NWIN
PAD_E = SHARD * 16
PAD_DST = 1 << 30

NRANGE = 4
RNG = 1152
SROWS = RNG + 16
CHROWS = 16
SHARD32 = PAD_E // 32
NW32 = SHARD32 // WIN
N0PAD = 4608
SENT0 = 4600

_MESH = plsc.VectorSubcoreMesh(core_axis_name="c", subcore_axis_name="s")


def _bcast(vec16, i):
    return vec16.at[jnp.full((16,), i, jnp.int32)].get(
        mode='promise_in_bounds')
_SC_PARAMS = pltpu.CompilerParams(needs_layout_passes=False)
_TC_PARAMS = pltpu.CompilerParams(vmem_limit_bytes=110 * 2**20)


def _matmul_bias_body(x_ref, w_ref, b_ref, o_ref):
    o_ref[...] = jnp.dot(x_ref[...], w_ref[...],
                         preferred_element_type=jnp.float32) + b_ref[...]


def _matmul_bias(x, w, b):
    m = x.shape[0]
    n = w.shape[1]
    return pl.pallas_call(
        _matmul_bias_body,
        out_shape=jax.ShapeDtypeStruct((m, n), jnp.float32),
    )(x, w, b[None, :])


def _matmul_body_nobias(x_ref, w_ref, o_ref):
    o_ref[...] = jnp.dot(x_ref[...], w_ref[...],
                         preferred_element_type=jnp.float32)


def _t2_body(x_ref, w_ref, asf_ref, adf_ref, wef_ref, aef_ref, ea_ref,
             xp_ref, tab_ref, wc_ref):
    x = x_ref[...]
    W = w_ref[...]
    xp_ref[...] = jnp.dot(x, W, preferred_element_type=jnp.float32)
    ki = lax.broadcasted_iota(jnp.int32, (DD, HEADS), 0)
    hi = lax.broadcasted_iota(jnp.int32, (DD, HEADS), 1)
    M = (ki // H2 == hi).astype(jnp.float32)
    Was = jnp.dot(W * asf_ref[...], M, precision=lax.Precision.HIGHEST,
                  preferred_element_type=jnp.float32)
    Wad = jnp.dot(W * adf_ref[...], M, precision=lax.Precision.HIGHEST,
                  preferred_element_type=jnp.float32)
    a_src = jnp.dot(x, Was, precision=lax.Precision.HIGHEST,
                    preferred_element_type=jnp.float32)
    a_dst = jnp.dot(x, Wad, precision=lax.Precision.HIGHEST,
                    preferred_element_type=jnp.float32)
    tab_ref[...] = jnp.concatenate([a_src, a_dst], axis=1)
    w8 = jnp.dot(wef_ref[...] * aef_ref[...], M,
                 precision=lax.Precision.HIGHEST,
                 preferred_element_type=jnp.float32)
    ea = ea_ref[...]
    ea_max = jnp.maximum(jnp.max(ea), 0.0)
    ea_min = jnp.minimum(jnp.min(ea), 0.0)
    ub = (jnp.max(a_src, axis=0, keepdims=True)
          + jnp.max(a_dst, axis=0, keepdims=True)
          + jnp.maximum(w8 * ea_max, w8 * ea_min))
    C = jnp.where(ub > 0, ub, 0.2 * ub)
    row = jnp.concatenate([w8, C], axis=1)
    wc_ref[...] = jnp.broadcast_to(row, (8, 16))


def _t2(x, W, asf, adf, wef, aef, ea2d):
    n = x.shape[0]
    return pl.pallas_call(
        _t2_body,
        compiler_params=_TC_PARAMS,
        out_shape=(
            jax.ShapeDtypeStruct((n, DD), jnp.float32),
            jax.ShapeDtypeStruct((n, 16), jnp.float32),
            jax.ShapeDtypeStruct((8, 16), jnp.float32),
        ),
    )(x, W, asf[None, :], adf[None, :], wef[None, :], aef[None, :], ea2d)


def _t3a_body(asum_ref, scale_ref, asc_ref):
    a = asum_ref[...]
    ji = lax.broadcasted_iota(jnp.int32, (HEADS, 32 * HEADS), 1)
    hi = lax.broadcasted_iota(jnp.int32, (HEADS, 32 * HEADS), 0)
    S = (ji % HEADS == hi).astype(jnp.float32)
    tot = jnp.dot(S, a, precision=lax.Precision.HIGHEST,
                  preferred_element_type=jnp.float32)
    ssum = jnp.sum(tot / (tot + 1e-16))
    inv = 1.0 / (tot + 1e-16)
    hi2 = lax.broadcasted_iota(jnp.int32, (HEADS, DD), 0)
    ci2 = lax.broadcasted_iota(jnp.int32, (HEADS, DD), 1)
    M2 = (ci2 // H2 == hi2).astype(jnp.float32)
    scale_ref[...] = lax.dot_general(
        inv, M2, (((0,), (0,)), ((), ())),
        precision=lax.Precision.HIGHEST,
        preferred_element_type=jnp.float32)
    asc_ref[...] = jnp.broadcast_to(ssum.reshape(1, 1), (8, 16))


def _t3b_body(outp_ref, scale_ref, b_ref, nw_ref, nb_ref, nms_ref, x_ref):
    y = outp_ref[...][:NN] * scale_ref[...][:NN] + b_ref[...]
    mean = jnp.mean(y, axis=0, keepdims=True)
    out = y - nms_ref[...] * mean
    var = jnp.mean(out * out, axis=0, keepdims=True)
    x_ref[...] = jnp.maximum(
        out / jnp.sqrt(var + 1e-5) * nw_ref[...] + nb_ref[...], 0.0)


_CBLK = 256


def _t3(outp, asum2d, b, nw, nb, nms):
    scale, asc = pl.pallas_call(
        _t3a_body,
        out_shape=(
            jax.ShapeDtypeStruct((N0PAD, DD), jnp.float32),
            jax.ShapeDtypeStruct((8, 16), jnp.float32),
        ),
    )(asum2d)
    x = pl.pallas_call(
        _t3b_body,
        grid=(DD // _CBLK,),
        in_specs=[
            pl.BlockSpec((N0PAD, _CBLK), lambda i: (0, i)),
            pl.BlockSpec((N0PAD, _CBLK), lambda i: (0, i)),
            pl.BlockSpec((1, _CBLK), lambda i: (0, i)),
            pl.BlockSpec((1, _CBLK), lambda i: (0, i)),
            pl.BlockSpec((1, _CBLK), lambda i: (0, i)),
            pl.BlockSpec((1, _CBLK), lambda i: (0, i)),
        ],
        out_specs=pl.BlockSpec((NN, _CBLK), lambda i: (0, i)),
        out_shape=jax.ShapeDtypeStruct((NN, DD), jnp.float32),
    )(outp, scale, b[None, :], nw[None, :], nb[None, :], nms[None, :])
    return x, asc


def _tla_body(s_ref, c_ref, o_ref):
    ssum = jnp.sum(s_ref[...], axis=0, keepdims=True)
    csum = jnp.sum(c_ref[...], axis=0, keepdims=True)
    o_ref[...] = ssum / jnp.maximum(csum, 1.0)


def _tla(sums, cnts):
    return pl.pallas_call(
        _tla_body,
        out_shape=jax.ShapeDtypeStruct((1, N0PAD), jnp.float32),
    )(sums, cnts)


def _s0_body(dst_hbm, ea_hbm, parts_hbm, sums_p, cnts_p, meta_d, meta_e,
             sem):
    cid = lax.axis_index("c")
    sid = lax.axis_index("s")

    @pl.when(cid == 0)
    def _():
        def zi(i, _):
            sums_p[pl.ds(i * 16, 16)] = jnp.zeros((16,), jnp.float32)
            cnts_p[pl.ds(i * 16, 16)] = jnp.zeros((16,), jnp.float32)
            return 0
        lax.fori_loop(0, N0PAD // 16, zi, 0)

        def win_loop(w, _):
            wbase = sid * SHARD + w * WIN
            pltpu.sync_copy(dst_hbm.at[pl.ds(wbase, WIN)], meta_d)
            pltpu.sync_copy(ea_hbm.at[pl.ds(wbase, WIN)], meta_e)

            def chunk(c, _):
                epos = wbase + c * 16 + lax.iota(jnp.int32, 16)
                d16 = meta_d[pl.ds(c * 16, 16)]
                e16 = meta_e[pl.ds(c * 16, 16)]
                ok = epos < EE
                idx = jnp.where(ok, jnp.minimum(d16, SENT0), SENT0)
                plsc.addupdate_scatter(sums_p, [idx],
                                       jnp.where(ok, e16, 0.0))
                plsc.addupdate_scatter(cnts_p, [idx],
                                       jnp.where(ok, 1.0, 0.0))
                return 0
            lax.fori_loop(0, WIN // 16, chunk, 0)
            return 0
        lax.fori_loop(0, NWIN, win_loop, 0)

        pltpu.sync_copy(sums_p,
                        parts_hbm.at[pl.ds(sid * 2 * N0PAD, N0PAD)])
        pltpu.sync_copy(cnts_p,
                        parts_hbm.at[pl.ds(sid * 2 * N0PAD + N0PAD, N0PAD)])


def _s0(dstp, eap0):
    return pl.kernel(
        _s0_body,
        out_type=jax.ShapeDtypeStruct((16 * 2 * N0PAD,), jnp.float32),
        mesh=_MESH,
        compiler_params=_SC_PARAMS,
        scratch_types=[
            pltpu.VMEM((N0PAD,), jnp.float32),
            pltpu.VMEM((N0PAD,), jnp.float32),
            pltpu.VMEM((WIN,), jnp.int32),
            pltpu.VMEM((WIN,), jnp.float32),
            pltpu.SemaphoreType.DMA,
        ],
    )(dstp, eap0)


def _s1a_body(src_hbm, dst_hbm, ea_hbm, tab_hbm, wc_hbm,
              p_hbm, asum_hbm,
              tab, wc_v, meta_s, meta_d, meta_e, pwin, asum_p, sem):
    cid = lax.axis_index("c")
    sid = lax.axis_index("s")
    wid = sid * 2 + cid
    pltpu.sync_copy(tab_hbm, tab)
    pltpu.sync_copy(wc_hbm, wc_v)
    wrow = wc_v[0]

    def za(i, _):
        asum_p[pl.ds(i * 16, 16)] = jnp.zeros((16,), jnp.float32)
        return 0
    lax.fori_loop(0, (8 * N0PAD) // 16, za, 0)

    def win_loop(w, _):
        wbase = wid * SHARD32 + w * WIN
        pltpu.sync_copy(src_hbm.at[pl.ds(wbase, WIN)], meta_s)
        pltpu.sync_copy(dst_hbm.at[pl.ds(wbase, WIN)], meta_d)
        pltpu.sync_copy(ea_hbm.at[pl.ds(wbase, WIN)], meta_e)

        def chunk(c, _):
            epos = wbase + c * 16 + lax.iota(jnp.int32, 16)
            ok = epos < EFULL
            s16 = jnp.clip(meta_s[pl.ds(c * 16, 16)], 0, NN - 1)
            d16r = meta_d[pl.ds(c * 16, 16)]
            d16 = jnp.clip(d16r, 0, NN - 1)
            e16 = meta_e[pl.ds(c * 16, 16)]
            dsum = jnp.minimum(d16r, SENT0)
            for h in range(HEADS):
                as_h = plsc.load_gather(tab, [s16 * 16 + h])
                ad_h = plsc.load_gather(tab, [d16 * 16 + 8 + h])
                z = as_h + ad_h + e16 * _bcast(wrow, h)
                z = jnp.where(z > 0, z, 0.2 * z)
                p = jnp.exp(z - _bcast(wrow, 8 + h))
                p = jnp.where(ok, p, 0.0)
                plsc.addupdate_scatter(asum_p, [h * N0PAD + dsum], p)
                pwin[pl.ds(h * WIN + c * 16, 16)] = p
            return 0
        lax.fori_loop(0, WIN // 16, chunk, 0)
        for h in range(HEADS):
            pltpu.sync_copy(pwin.at[pl.ds(h * WIN, WIN)],
                            p_hbm.at[pl.ds(h * PAD_E + wbase, WIN)])
        return 0
    lax.fori_loop(0, NW32, win_loop, 0)
    pltpu.sync_copy(asum_p,
                    asum_hbm.at[pl.ds(wid * 8 * N0PAD, 8 * N0PAD)])


def _s1a(srcp, dstp, eap, tabflat, wc):
    return pl.kernel(
        _s1a_body,
        out_type=(
            jax.ShapeDtypeStruct((PAD_E * 8,), jnp.float32),
            jax.ShapeDtypeStruct((32 * 8 * N0PAD,), jnp.float32),
        ),
        mesh=_MESH,
        compiler_params=_SC_PARAMS,
        scratch_types=[
            pltpu.VMEM((NN * 16,), jnp.float32),
            pltpu.VMEM((8, 16), jnp.float32),
            pltpu.VMEM((WIN,), jnp.int32),
            pltpu.VMEM((WIN,), jnp.int32),
            pltpu.VMEM((WIN,), jnp.float32),
            pltpu.VMEM((WIN * 8,), jnp.float32),
            pltpu.VMEM((8 * N0PAD,), jnp.float32),
            pltpu.SemaphoreType.DMA,
        ],
    )(srcp, dstp, eap, tabflat, wc)


SEG = 2368


def _s1b_body(src_hbm, xp_hbm, rows_hbm, meta_s, gsrc, rowbuf, sem):
    cid = lax.axis_index("c")
    sid = lax.axis_index("s")
    wid = sid * 2 + cid

    for seg in range(SHARD32 // SEG):
        segbase = wid * SHARD32 + seg * SEG
        pltpu.sync_copy(src_hbm.at[pl.ds(segbase, SEG)], meta_s)

        def chunk(c, _):
            for q in range(4):
                j0 = c * 64 + q * 16
                s16 = jnp.clip(meta_s[pl.ds(j0, 16)], 0, NN - 1)
                gsrc[pl.ds(q * 16, 16)] = s16
            pltpu.async_copy(xp_hbm.at[gsrc], rowbuf, sem).wait()
            pltpu.sync_copy(rowbuf,
                            rows_hbm.at[pl.ds(segbase + c * 64, 64)])
            return 0
        lax.fori_loop(0, SEG // 64, chunk, 0)


def _s1b(srcp, xp):
    return pl.kernel(
        _s1b_body,
        out_type=jax.ShapeDtypeStruct((PAD_E, DD), jnp.float32),
        mesh=_MESH,
        compiler_params=_SC_PARAMS,
        scratch_types=[
            pltpu.VMEM((SEG,), jnp.int32),
            pltpu.VMEM((64,), jnp.int32),
            pltpu.VMEM((64, DD), jnp.float32),
            pltpu.SemaphoreType.DMA,
        ],
    )(srcp, xp)


EBLK = 256
NEB = PAD_E // EBLK


def _t4agg_body(dst_ref, p_ref, rows_ref, out_ref):
    i = pl.program_id(0)

    @pl.when(i == 0)
    def _():
        out_ref[...] = jnp.zeros(out_ref.shape, jnp.float32)

    dv = dst_ref[0, 0]
    ni = lax.broadcasted_iota(jnp.int32, (EBLK, N0PAD), 1)
    oneh = (dv[:, None] == ni).astype(jnp.float32)
    hi2 = lax.broadcasted_iota(jnp.int32, (HEADS, DD), 0)
    ci2 = lax.broadcasted_iota(jnp.int32, (HEADS, DD), 1)
    M2 = (ci2 // H2 == hi2).astype(jnp.float32)
    smat = lax.dot_general(p_ref[...], M2, (((0,), (0,)), ((), ())),
                           precision=lax.Precision.HIGHEST,
                           preferred_element_type=jnp.float32)
    rb = rows_ref[...] * smat
    out_ref[...] += lax.dot_general(
        oneh, rb, (((0,), (0,)), ((), ())),
        precision=lax.Precision.HIGHEST,
        preferred_element_type=jnp.float32)


def _t4agg(dst3, p8, rows):
    return pl.pallas_call(
        _t4agg_body,
        compiler_params=_TC_PARAMS,
        grid=(NEB,),
        in_specs=[
            pl.BlockSpec((1, 1, EBLK), lambda i: (i, 0, 0)),
            pl.BlockSpec((HEADS, EBLK), lambda i: (0, i)),
            pl.BlockSpec((EBLK, DD), lambda i: (i, 0)),
        ],
        out_specs=pl.BlockSpec((N0PAD, DD), lambda i: (0, 0)),
        out_shape=jax.ShapeDtypeStruct((N0PAD, DD), jnp.float32),
    )(dst3, p8, rows)


def _att_body(src_hbm, dst_hbm, mrow_hbm, out_hbm,
              zbuf, meta_s, meta_d, idxbuf, constbuf, mrow_v, sem):
    cid = lax.axis_index("c")
    sid = lax.axis_index("s")
    half = NN // 2
    row_lo = cid * half

    flat_base = row_lo * NN
    flat_len = half * NN
    base_len = (flat_len // 16) & ~7
    extra = flat_len - base_len * 16

    ZCH = zbuf.shape[0]

    def zinit(i, _):
        zbuf[pl.ds(i * 16, 16)] = jnp.zeros((16,), jnp.float32)
        return 0
    lax.fori_loop(0, ZCH // 16, zinit, 0)

    tstart = flat_base + sid * base_len
    nfull = base_len // ZCH
    rem = base_len - nfull * ZCH

    def zloop(i, _):
        pltpu.sync_copy(zbuf, out_hbm.at[pl.ds(tstart + i * ZCH, ZCH)])
        return 0
    lax.fori_loop(0, nfull, zloop, 0)
    if rem:
        pltpu.sync_copy(zbuf.at[pl.ds(0, rem)],
                        out_hbm.at[pl.ds(tstart + nfull * ZCH, rem)])
    if extra:
        @pl.when(sid == 0)
        def _():
            pltpu.sync_copy(zbuf.at[pl.ds(0, extra)],
                            out_hbm.at[pl.ds(flat_base + 16 * base_len, extra)])

    plsc.subcore_barrier()

    pltpu.sync_copy(mrow_hbm, mrow_v)
    mvec = mrow_v[0]
    shard_base = sid * SHARD

    def win_loop(w, _):
        wbase = shard_base + w * WIN
        pltpu.sync_copy(src_hbm.at[pl.ds(wbase, WIN)], meta_s)
        pltpu.sync_copy(dst_hbm.at[pl.ds(wbase, WIN)], meta_d)

        def chunk(c, _):
            s16 = meta_s[pl.ds(c * 16, 16)]
            d16 = meta_d[pl.ds(c * 16, 16)]
            ok = jnp.logical_and(
                d16 < NN,
                jnp.logical_and(s16 >= row_lo, s16 < row_lo + half))
            epos = wbase + c * 16 + lax.iota(jnp.int32, 16)
            dummy = (row_lo + (epos & 2047)) * (NN + 1)
            flat = jnp.where(ok, s16 * NN + d16, dummy)
            idxbuf[pl.ds(c * 16, 16)] = flat
            constbuf[pl.ds(c * 16, 16)] = mvec
            return 0
        lax.fori_loop(0, WIN // 16, chunk, 0)
        pltpu.sync_copy(constbuf, out_hbm.at[idxbuf])
        return 0
    lax.fori_loop(0, NWIN, win_loop, 0)


def _att_scatter(srcp, dstp, mrow):
    return pl.kernel(
        _att_body,
        out_type=jax.ShapeDtypeStruct((NN * NN,), jnp.float32),
        mesh=_MESH,
        compiler_params=_SC_PARAMS,
        scratch_types=[
            pltpu.VMEM((16384,), jnp.float32),
            pltpu.VMEM((WIN,), jnp.int32),
            pltpu.VMEM((WIN,), jnp.int32),
            pltpu.VMEM((WIN,), jnp.int32),
            pltpu.VMEM((WIN,), jnp.float32),
            pltpu.VMEM((8, 16), jnp.float32),
            pltpu.SemaphoreType.DMA,
        ],
    )(srcp, dstp, mrow)


B = 1024
_BPW = B // 32


def _fin_body(y_hbm, idxd_hbm, idxc_hbm, brow_hbm, out_hbm,
              ytab, idxd_v, idxc_v, out_v, brow_v, sem):
    cid = lax.axis_index("c")
    sid = lax.axis_index("s")
    wid = sid * 2 + cid
    base = wid * _BPW
    pltpu.sync_copy(brow_hbm, brow_v)
    bias = brow_v[0]
    pltpu.sync_copy(y_hbm, ytab)
    pltpu.sync_copy(idxd_hbm.at[pl.ds(base, _BPW)], idxd_v)
    pltpu.sync_copy(idxc_hbm.at[pl.ds(base, _BPW)], idxc_v)

    def chunk(c, _):
        d16 = idxd_v[pl.ds(c * 16, 16)]
        c16 = idxc_v[pl.ds(c * 16, 16)]
        yd = plsc.load_gather(ytab, [d16 * 16])
        yc = plsc.load_gather(ytab, [c16 * 16 + 1])
        out_v[pl.ds(c * 16, 16)] = yd + yc + bias
        return 0
    lax.fori_loop(0, _BPW // 16, chunk, 0)
    pltpu.sync_copy(out_v, out_hbm.at[pl.ds(base, _BPW)])


def _final_gather(y_flat, idx_drug, idx_cell, brow):
    return pl.kernel(
        _fin_body,
        out_type=jax.ShapeDtypeStruct((B,), jnp.float32),
        mesh=_MESH,
        compiler_params=_SC_PARAMS,
        scratch_types=[
            pltpu.VMEM((NN * 16,), jnp.float32),
            pltpu.VMEM((_BPW,), jnp.int32),
            pltpu.VMEM((_BPW,), jnp.int32),
            pltpu.VMEM((_BPW,), jnp.float32),
            pltpu.VMEM((8, 16), jnp.float32),
            pltpu.SemaphoreType.DMA,
        ],
    )(y_flat, idx_drug, idx_cell, brow)


def kernel(drug, cell, gene, edge_attr, edge_index, idx_drug, idx_cell,
           params):
    p = params
    x0 = jnp.concatenate([
        _matmul_bias(drug, p['Wd'], p['bd']),
        _matmul_bias(cell, p['Wc'], p['bc']),
        _matmul_bias(gene, p['Wg'], p['bg']),
    ], axis=0)
    ea0 = edge_attr.astype(jnp.float32)
    loop = jnp.arange(NN, dtype=jnp.int32)
    src = jnp.concatenate([edge_index[0].astype(jnp.int32), loop])
    dst = jnp.concatenate([edge_index[1].astype(jnp.int32), loop])
    npad = PAD_E - EFULL
    srcp = jnp.concatenate([src, jnp.zeros((npad,), jnp.int32)])
    dstp = jnp.concatenate([dst, jnp.full((npad,), PAD_DST, jnp.int32)])
    eap0 = jnp.concatenate([ea0, jnp.zeros((PAD_E - EE,), jnp.float32)])

    parts = _s0(dstp, eap0).reshape(16, 2, N0PAD)
    la = _tla(parts[:, 0], parts[:, 1])[0, :NN]
    dst3 = dstp.reshape(PAD_E // EBLK, 1, EBLK)
    eap = jnp.concatenate([ea0, la, jnp.zeros((npad,), jnp.float32)])
    ea2d = ea0.reshape(1125, 128)

    def layer(x, pre):
        asf = p[pre + '_as'].reshape(DD)
        adf = p[pre + '_ad'].reshape(DD)
        wef = p[pre + '_We'].reshape(DD)
        aef = p[pre + '_ae'].reshape(DD)
        xp, tab, wc = _t2(x, p[pre + '_W'], asf, adf, wef, aef, ea2d)
        pfull, asumf = _s1a(srcp, dstp, eap, tab.reshape(NN * 16), wc)
        rows = _s1b(srcp, xp)
        outp = _t4agg(dst3, pfull.reshape(8, PAD_E), rows)
        return outp, asumf.reshape(32 * 8, N0PAD)

    outp1, asum1 = layer(x0, 'g1')
    x1, ss1 = _t3(outp1, asum1, p['g1_b'], p['n1_w'], p['n1_b'], p['n1_ms'])
    outp2, asum2 = layer(x1, 'g2')
    x2, ss2 = _t3(outp2, asum2, p['g2_b'], p['n2_w'], p['n2_b'], p['n2_ms'])

    m = (ss1[0, 0] + ss2[0, 0]) / (EFULL * HEADS)
    att = _att_scatter(srcp, dstp, jnp.full((8, 16), m, jnp.float32))
    att = att.reshape(NN, NN)

    HH = HEADS * H3
    L2 = jnp.zeros((HH, 16), jnp.float32)
    L2 = L2.at[:, 0].set(p['l1_W'][:HH, 0])
    L2 = L2.at[:, 1].set(p['l1_W'][HH:, 0])
    y = pl.pallas_call(
        _matmul_body_nobias,
        out_shape=jax.ShapeDtypeStruct((NN, 16), jnp.float32),
    )(x2, L2)
    brow = jnp.full((8, 16), p['l1_b'][0], jnp.float32)
    out = _final_gather(y.reshape(NN * 16), idx_drug.astype(jnp.int32),
                        idx_cell.astype(jnp.int32), brow)
    return out[:, None], att
